# baseline (speedup 1.0000x reference)
import functools

import jax
import jax.numpy as jnp
from jax import lax
from jax.experimental import pallas as pl
from jax.experimental.pallas import tpu as pltpu
from jax.experimental.pallas import tpu_sc as plsc

B = 16384
NCAT = 26
V = 100000
D = 16
NNUM = 13
EMB = NCAT * D
EPS = 1e-5

_NC = 2
_NS = 16
_NW = _NC * _NS

_TOTAL = B * NCAT
_PER_W = _TOTAL // _NW
_CHUNK = 1664
_NCHUNK = _PER_W // _CHUNK

TB0 = 1024
TB = 512
NT = B // TB


_CH = 416
_NCH = _PER_W // _CH
_GROUPS = _CH // 16
_OUTW = 512


_BPW = _PER_W // NCAT


def _sc_gather(xcat_t, tab128):
    mesh = plsc.VectorSubcoreMesh(core_axis_name="c", subcore_axis_name="s")

    @functools.partial(
        pl.kernel,
        mesh=mesh,
        compiler_params=pltpu.CompilerParams(needs_layout_passes=False),
        out_type=jax.ShapeDtypeStruct((B, _OUTW), jnp.float32),
        scratch_types=[
            pltpu.VMEM((NCAT, 128), jnp.int32),
            pltpu.VMEM((_CH,), jnp.int32),
            pltpu.VMEM((_CH,), jnp.int32),
            pltpu.VMEM((_CH,), jnp.int32),
            pltpu.VMEM((_CH,), jnp.int32),
            pltpu.VMEM((2, _CH, 128), jnp.float32),
            pltpu.VMEM((2, 16, _OUTW), jnp.float32),
            pltpu.SemaphoreType.DMA((2,)),
            pltpu.SemaphoreType.DMA((2,)),
        ],
    )
    def gather_kernel(xcat_h, tab_h, out_h,
                      xbuf, gbuf0, gbuf1, pbuf0, pbuf1, rbuf, obuf,
                      rsem, osem):
        gbufs = (gbuf0, gbuf1)
        pbufs = (pbuf0, pbuf1)
        wid = lax.axis_index("s") * _NC + lax.axis_index("c")
        brow = pl.multiple_of(wid * _BPW, 8)

        lanes = lax.iota(jnp.int32, 16)

        def load_block(ch):
            c0 = pl.multiple_of(brow + (ch // 8) * 128, 128)
            pltpu.sync_copy(xcat_h.at[:, pl.ds(c0, 128)], xbuf)

        def build_idx(ch, s):
            def grp(g, _):
                jv = g * 16 + lanes
                cv = jv % NCAT
                bv = (ch % 8) * 16 + jv // NCAT
                xv = plsc.load_gather(xbuf, [cv, bv])
                flat = cv * V + xv
                gbufs[s][pl.ds(g * 16, 16)] = lax.shift_right_logical(flat, 3)
                pbufs[s][pl.ds(g * 16, 16)] = (flat & 7) * D
                return 0

            lax.fori_loop(0, _GROUPS, grp, 0)

        def gath_start(s):
            pltpu.async_copy(tab_h.at[gbufs[s]], rbuf.at[s], rsem.at[s])

        def gath_wait(s):
            pltpu.make_async_copy(tab_h.at[gbufs[s]], rbuf.at[s],
                                  rsem.at[s]).wait()

        def out_ref(ch, s):
            r0 = pl.multiple_of(brow + ch * 16, 8)
            return out_h.at[pl.ds(r0, 16), :]

        def out_start(ch, s):
            pltpu.async_copy(obuf.at[s], out_ref(ch, s), osem.at[s])

        def out_wait(ch, s):
            pltpu.make_async_copy(obuf.at[s], out_ref(ch, s),
                                  osem.at[s]).wait()

        def extract(s):
            def grp(g, _):
                pv = pbufs[s][pl.ds(g * 16, 16)]
                rowv = g * 16 + lanes
                sr = rowv // NCAT
                sc = (rowv % NCAT) * D
                for d in range(16):
                    v = plsc.load_gather(rbuf.at[s], [rowv, pv + d])
                    plsc.store_scatter(obuf.at[s], [sr, sc + d], v)
                return 0

            lax.fori_loop(0, _GROUPS, grp, 0)

        load_block(0)
        build_idx(0, 0)
        gath_start(0)
        build_idx(1, 1)
        gath_start(1)
        gath_wait(0)
        extract(0)
        out_start(0, 0)
        build_idx(2, 0)
        gath_start(0)
        gath_wait(1)
        extract(1)
        out_start(1, 1)
        build_idx(3, 1)
        gath_start(1)

        def step(k, _):
            for par in range(2):
                ch = 2 * k + par
                gath_wait(par)
                out_wait(ch - 2, par)
                extract(par)
                out_start(ch, par)

                @pl.when(ch + 2 < _NCH)
                def _():
                    @pl.when((ch + 2) % 8 == 0)
                    def _():
                        load_block(ch + 2)

                    build_idx(ch + 2, par)
                    gath_start(par)

            return 0

        lax.fori_loop(1, _NCH // 2, step, 0)
        out_wait(_NCH - 2, 0)
        out_wait(_NCH - 1, 1)

    return gather_kernel(xcat_t, tab128)


def _leaky(t):
    return jnp.where(t >= 0, t, 0.01 * t)


def _bn_ac(s, q, gamma, beta):
    mu = s * (1.0 / B)
    var = jnp.maximum(q * (1.0 / B) - mu * mu, 0.0)
    a = gamma * lax.rsqrt(var + EPS)
    return a, beta - mu * a


def _stats_body(x_ref, s_ref, q_ref):
    x = x_ref[...]

    @pl.when(pl.program_id(0) == 0)
    def _():
        s_ref[...] = jnp.zeros_like(s_ref)
        q_ref[...] = jnp.zeros_like(q_ref)

    s_ref[...] += jnp.sum(x, axis=0, keepdims=True)
    q_ref[...] += jnp.sum(x * x, axis=0, keepdims=True)


def _stats(x, tb):
    bb, h = x.shape
    return pl.pallas_call(
        _stats_body,
        grid=(bb // tb,),
        in_specs=[pl.BlockSpec((tb, h), lambda i: (i, 0))],
        out_specs=[
            pl.BlockSpec((1, h), lambda i: (0, 0)),
            pl.BlockSpec((1, h), lambda i: (0, 0)),
        ],
        out_shape=[
            jax.ShapeDtypeStruct((1, h), jnp.float32),
            jax.ShapeDtypeStruct((1, h), jnp.float32),
        ],
    )(x)


def _layer0_body(e_ref, x_ref, s0_ref, q0_ref, g_ref, bt_ref,
                 we_ref, wn_ref, b_ref, y_ref, s_ref, q_ref):
    a0, c0 = _bn_ac(s0_ref[...], q0_ref[...], g_ref[...], bt_ref[...])
    xn = x_ref[...] * a0 + c0
    t = jnp.dot(e_ref[...][:, :EMB], we_ref[...],
                preferred_element_type=jnp.float32)
    t += jnp.dot(xn, wn_ref[...], preferred_element_type=jnp.float32)
    y = _leaky(t + b_ref[...])
    y_ref[...] = y

    @pl.when(pl.program_id(0) == 0)
    def _():
        s_ref[...] = jnp.zeros_like(s_ref)
        q_ref[...] = jnp.zeros_like(q_ref)

    s_ref[...] += jnp.sum(y, axis=0, keepdims=True)
    q_ref[...] += jnp.sum(y * y, axis=0, keepdims=True)


def _layer0(emb, xnum, s0, q0, bn0g, bn0b, w_emb, w_num, b):
    h = w_emb.shape[1]
    row = lambda v: v.reshape(1, -1)
    const = lambda shape: pl.BlockSpec(shape, lambda i: (0, 0))
    return pl.pallas_call(
        _layer0_body,
        grid=(B // TB0,),
        in_specs=[
            pl.BlockSpec((TB0, _OUTW), lambda i: (i, 0)),
            pl.BlockSpec((TB0, NNUM), lambda i: (i, 0)),
            const((1, NNUM)), const((1, NNUM)), const((1, NNUM)),
            const((1, NNUM)),
            const((EMB, h)), const((NNUM, h)), const((1, h)),
        ],
        out_specs=[
            pl.BlockSpec((TB0, h), lambda i: (i, 0)),
            const((1, h)),
            const((1, h)),
        ],
        out_shape=[
            jax.ShapeDtypeStruct((B, h), jnp.float32),
            jax.ShapeDtypeStruct((1, h), jnp.float32),
            jax.ShapeDtypeStruct((1, h), jnp.float32),
        ],
    )(emb, xnum, s0, q0, row(bn0g), row(bn0b), w_emb, w_num, row(b))


def _tail_body(y0_hbm, s_in, q_in,
               g0, bt0, w1, b1, g1, bt1,
               w2, b2, g2, bt2, w3, b3, g3, bt3,
               wout, bout,
               out_hbm,
               ybuf, obuf, y1, y2, y3,
               ysem, osem):

    def y0_load(i):
        return pltpu.make_async_copy(
            y0_hbm.at[pl.ds(i * TB, TB)], ybuf.at[i % 2], ysem.at[i % 2])

    a1, c1 = _bn_ac(s_in[...], q_in[...], g0[...], bt0[...])

    y0_load(0).start()
    s = jnp.zeros((1, 300), jnp.float32)
    q = jnp.zeros((1, 300), jnp.float32)
    for i in range(NT):
        if i + 1 < NT:
            y0_load(i + 1).start()
        y0_load(i).wait()
        x = ybuf[i % 2] * a1 + c1
        t = jnp.dot(x, w1[...], preferred_element_type=jnp.float32)
        y = _leaky(t + b1[...])
        y1[pl.ds(i * TB, TB), :] = y
        s = s + jnp.sum(y, axis=0, keepdims=True)
        q = q + jnp.sum(y * y, axis=0, keepdims=True)

    def layer(y_prev, y_next, w, b, a, c, h):
        s = jnp.zeros((1, h), jnp.float32)
        q = jnp.zeros((1, h), jnp.float32)
        for i in range(NT):
            x = y_prev[pl.ds(i * TB, TB), :] * a + c
            t = jnp.dot(x, w[...], preferred_element_type=jnp.float32)
            y = _leaky(t + b[...])
            y_next[pl.ds(i * TB, TB), :] = y
            s = s + jnp.sum(y, axis=0, keepdims=True)
            q = q + jnp.sum(y * y, axis=0, keepdims=True)
        return s, q

    a, c = _bn_ac(s, q, g1[...], bt1[...])
    s, q = layer(y1, y2, w2, b2, a, c, 200)
    a, c = _bn_ac(s, q, g2[...], bt2[...])
    s, q = layer(y2, y3, w3, b3, a, c, 100)
    a, c = _bn_ac(s, q, g3[...], bt3[...])

    def o_copy(i):
        return pltpu.make_async_copy(
            obuf.at[i % 2], out_hbm.at[pl.ds(i * TB, TB)], osem.at[i % 2])

    for i in range(NT):
        if i >= 2:
            o_copy(i - 2).wait()
        x = y3[pl.ds(i * TB, TB), :] * a + c
        obuf[i % 2] = (jnp.dot(x, wout[...], preferred_element_type=jnp.float32)
                       + bout[...])
        o_copy(i).start()
    o_copy(NT - 2).wait()
    o_copy(NT - 1).wait()


def _tail(y0, s, q, g0, bt0, w1, b1, g1, bt1,
          w2, b2, g2, bt2, w3, b3, g3, bt3, wout, bout):
    hbm = pl.BlockSpec(memory_space=pltpu.MemorySpace.HBM)
    vmem = pl.BlockSpec(memory_space=pltpu.MemorySpace.VMEM)
    row = lambda v: v.reshape(1, -1)
    return pl.pallas_call(
        _tail_body,
        in_specs=[hbm] + [vmem] * 18,
        out_specs=hbm,
        out_shape=jax.ShapeDtypeStruct((B, 1), jnp.float32),
        scratch_shapes=[
            pltpu.VMEM((2, TB, 600), jnp.float32),
            pltpu.VMEM((2, TB, 1), jnp.float32),
            pltpu.VMEM((B, 300), jnp.float32),
            pltpu.VMEM((B, 200), jnp.float32),
            pltpu.VMEM((B, 100), jnp.float32),
            pltpu.SemaphoreType.DMA((2,)),
            pltpu.SemaphoreType.DMA((2,)),
        ],
        compiler_params=pltpu.CompilerParams(
            vmem_limit_bytes=63 * 1024 * 1024),
    )(y0, s, q, row(g0), row(bt0), w1, row(b1), row(g1), row(bt1),
      w2, row(b2), row(g2), row(bt2), w3, row(b3), row(g3), row(bt3),
      wout, row(bout))


def kernel(x_categorical, x_numerical, tables, bn0_g, bn0_b,
           W0, b0, g0, beta0, W1, b1, g1, beta1,
           W2, b2, g2, beta2, W3, b3, g3, beta3, Wout, bout):
    tab128 = tables.reshape(NCAT * V // 8, 128)

    emb = _sc_gather(x_categorical.T, tab128)

    s0, q0 = _stats(x_numerical, 2048)
    y0, s, q = _layer0(emb, x_numerical, s0, q0, bn0_g, bn0_b,
                       W0[:EMB], W0[EMB:], b0)
    return _tail(y0, s, q, g0, beta0, W1, b1, g1, beta1,
                 W2, b2, g2, beta2, W3, b3, g3, beta3, Wout, bout)
